# baseline (speedup 1.0000x reference)
import functools

import jax
import jax.numpy as jnp
from jax import lax
from jax.experimental import pallas as pl
from jax.experimental.pallas import tpu as pltpu
from jax.experimental.pallas import tpu_sc as plsc

_NW = 32
_RB = 256
_K = 128
_GS = 2
_GD = 2


def _wid():
    return lax.axis_index("s") * 2 + lax.axis_index("c")


def _pair_table(V, D):
    nblk = V // _RB
    mesh = plsc.VectorSubcoreMesh(core_axis_name="c", subcore_axis_name="s")

    @functools.partial(
        pl.kernel,
        mesh=mesh,
        out_type=jax.ShapeDtypeStruct((V // 2, 2 * D), jnp.float32),
        scratch_types=[
            pltpu.VMEM((2, _RB, D), jnp.float32),
            pltpu.VMEM((2, _RB // 2, 2 * D), jnp.float32),
        ]
        + [pltpu.SemaphoreType.DMA] * 4,
    )
    def k1(w_hbm, w2_hbm, buf_v, pbuf_v, *sems):
        lsem = sems[:2]
        wsem = sems[2:]
        w = _wid()

        def blk_of(i):
            return w + i * _NW

        def fire_load(i, sl):
            pltpu.async_copy(
                w_hbm.at[pl.ds(blk_of(i) * _RB, _RB)], buf_v.at[sl], lsem[sl]
            )

        fire_load(0, 0)

        def repack(sl, npairs):
            def rloop(rp, c):
                for parity in range(2):
                    for q in range(D // 16):
                        x = buf_v[sl, 2 * rp + parity, pl.ds(16 * q, 16)]
                        pbuf_v[sl, rp, pl.ds(parity * D + 16 * q, 16)] = x
                return c

            lax.fori_loop(0, npairs, rloop, 0)

        def step(i, sl):
            pltpu.make_async_copy(
                w_hbm.at[pl.ds(0, _RB)], buf_v.at[sl], lsem[sl]
            ).wait()

            @pl.when(blk_of(i + 1) < nblk)
            def _():
                fire_load(i + 1, 1 - sl)

            @pl.when(i >= 2)
            def _():
                pltpu.make_async_copy(
                    pbuf_v.at[sl], w2_hbm.at[pl.ds(0, _RB // 2)], wsem[sl]
                ).wait()

            repack(sl, _RB // 2)
            pltpu.async_copy(
                pbuf_v.at[sl],
                w2_hbm.at[pl.ds(blk_of(i) * (_RB // 2), _RB // 2)],
                wsem[sl],
            )

        def body(i0, carry):
            for k in range(2):
                i = i0 * 2 + k

                @pl.when(blk_of(i) < nblk)
                def _():
                    step(i, k)

            return carry

        nb_max = nblk // _NW + (1 if nblk % _NW else 0)
        lax.fori_loop(0, (nb_max + 1) // 2, body, 0)
        for sl in range(2):
            pltpu.make_async_copy(
                pbuf_v.at[sl], w2_hbm.at[pl.ds(0, _RB // 2)], wsem[sl]
            ).wait()

        tail = V % _RB
        if tail:
            @pl.when(w == _NW - 1)
            def _():
                t0 = nblk * _RB
                pltpu.sync_copy(
                    w_hbm.at[pl.ds(t0, tail)], buf_v.at[0, pl.ds(0, tail)]
                )
                repack(0, tail // 2)
                pltpu.sync_copy(
                    pbuf_v.at[0, pl.ds(0, tail // 2)],
                    w2_hbm.at[pl.ds(t0 // 2, tail // 2)],
                )

    return k1


def _gather_rows(B, D):
    steps = B // (_NW * _K)
    mesh = plsc.VectorSubcoreMesh(core_axis_name="c", subcore_axis_name="s")

    @functools.partial(
        pl.kernel,
        mesh=mesh,
        out_type=jax.ShapeDtypeStruct((B, D), jnp.float32),
        scratch_types=[
            pltpu.VMEM((steps, _K), jnp.int32),
            pltpu.VMEM((steps, _K), jnp.int32),
            pltpu.VMEM((_GS, _K, 2 * D), jnp.float32),
            pltpu.VMEM((2, _K, D), jnp.float32),
        ]
        + [pltpu.SemaphoreType.DMA] * (_GS + 2),
    )
    def k2(pidx_hbm, par_hbm, w2_hbm, out_hbm, pidx_v, par_v, rows_v, wbuf_v, *sems):
        gsem, wsem = sems[:_GS], sems[_GS:]
        w = _wid()
        pltpu.sync_copy(pidx_hbm.at[w], pidx_v)
        pltpu.sync_copy(par_hbm.at[w], par_v)
        base = w * (steps * _K)

        def fire_gather(j, b):
            pltpu.async_copy(w2_hbm.at[pidx_v.at[j]], rows_v.at[b], gsem[b])

        for b in range(_GD):
            fire_gather(b, b % _GS)

        def compact(j, gs, ts):
            def gloop(g, c):
                pvec = par_v[j, pl.ds(16 * g, 16)]
                for lane in range(16):
                    half = pvec[lane]
                    r = g * 16 + lane
                    for q in range(D // 16):
                        x = rows_v[gs, r, pl.ds(half + 16 * q, 16)]
                        wbuf_v[ts, r, pl.ds(16 * q, 16)] = x
                return c

            lax.fori_loop(0, _K // 16, gloop, 0)

        def step(j, gs, ngs, ts):
            pltpu.make_async_copy(
                w2_hbm.at[pidx_v.at[0]], rows_v.at[gs], gsem[gs]
            ).wait()

            @pl.when(j >= 2)
            def _():
                pltpu.make_async_copy(
                    wbuf_v.at[ts], out_hbm.at[pl.ds(0, _K)], wsem[ts]
                ).wait()

            compact(j, gs, ts)
            pltpu.async_copy(
                wbuf_v.at[ts], out_hbm.at[pl.ds(base + j * _K, _K)], wsem[ts]
            )

            @pl.when(j + _GD < steps)
            def _():
                fire_gather(j + _GD, ngs)

        def body(j0, carry):
            for k in range(4):
                j = j0 * 4 + k
                step(j, k % _GS, (k + _GD) % _GS, k & 1)
            return carry

        lax.fori_loop(0, steps // 4, body, 0)
        for ts in range(2):
            pltpu.make_async_copy(
                wbuf_v.at[ts], out_hbm.at[pl.ds(0, _K)], wsem[ts]
            ).wait()

    return k2


def kernel(tokens, weight):
    S, T = tokens.shape
    V, D = weight.shape
    B = S * T
    steps = B // (_NW * _K)
    idx = tokens.reshape(_NW, steps, _K).astype(jnp.int32)
    pidx = idx >> 1
    par = (idx & 1) * D
    w2 = _pair_table(V, D)(weight)
    out = _gather_rows(B, D)(pidx, par, w2)
    return out.reshape(S, T, D)


# speedup vs baseline: 1.1574x; 1.1574x over previous
import functools

import jax
import jax.numpy as jnp
from jax import lax
from jax.experimental import pallas as pl
from jax.experimental.pallas import tpu as pltpu
from jax.experimental.pallas import tpu_sc as plsc

_NW = 32
_K = 128
_SLOTS = 8
_DEPTH = 4


def _embed_lookup(idx, weight, steps):
    B = _NW * steps * _K
    D = weight.shape[1]
    mesh = plsc.VectorSubcoreMesh(core_axis_name="c", subcore_axis_name="s")

    @functools.partial(
        pl.kernel,
        mesh=mesh,
        compiler_params=pltpu.CompilerParams(use_tc_tiling_on_sc=False),
        out_type=jax.ShapeDtypeStruct((B, D), jnp.float32),
        scratch_types=[
            pltpu.VMEM((steps, _K), jnp.int32),
            pltpu.VMEM((_SLOTS, _K, D), jnp.float32),
        ]
        + [pltpu.SemaphoreType.DMA] * (2 * _SLOTS),
    )
    def k(idx_hbm, w_hbm, out_hbm, idx_v, rows_v, *sems):
        gsem, wsem = sems[:_SLOTS], sems[_SLOTS:]
        wid = lax.axis_index("s") * 2 + lax.axis_index("c")
        pltpu.sync_copy(idx_hbm.at[wid], idx_v)
        base = wid * (steps * _K)

        def fire_gather(j, b):
            pltpu.async_copy(w_hbm.at[idx_v.at[j]], rows_v.at[b], gsem[b])

        for b in range(_DEPTH):
            fire_gather(b, b)

        def outer(j0, carry):
            for b in range(_SLOTS):
                j = j0 * _SLOTS + b
                pltpu.make_async_copy(
                    w_hbm.at[idx_v.at[j]], rows_v.at[b], gsem[b]
                ).wait()
                pltpu.async_copy(
                    rows_v.at[b], out_hbm.at[pl.ds(base + j * _K, _K)], wsem[b]
                )
                jn = j + _DEPTH
                bn = (b + _DEPTH) % _SLOTS

                @pl.when(jn < steps)
                def _():
                    @pl.when(jn >= _SLOTS)
                    def _():
                        pltpu.make_async_copy(
                            rows_v.at[bn],
                            out_hbm.at[pl.ds(base + (jn - _SLOTS) * _K, _K)],
                            wsem[bn],
                        ).wait()

                    fire_gather(jn, bn)

            return carry

        lax.fori_loop(0, steps // _SLOTS, outer, 0)
        for b in range(_SLOTS):
            pltpu.make_async_copy(
                rows_v.at[b], out_hbm.at[pl.ds(base, _K)], wsem[b]
            ).wait()

    return k(idx, weight)


def kernel(tokens, weight):
    S, T = tokens.shape
    D = weight.shape[1]
    B = S * T
    steps = B // (_NW * _K)
    idx = tokens.reshape(_NW, steps, _K).astype(jnp.int32)
    out = _embed_lookup(idx, weight, steps)
    return out.reshape(S, T, D)
